# baseline (device time: 89648 ns/iter reference)
import jax
import jax.numpy as jnp
from jax import lax
from jax.experimental import pallas as pl
from jax.experimental.pallas import tpu as pltpu

N_DEV = 16
CROWS = 512
FWD = N_DEV // 2
BWD = N_DEV - 1 - FWD


def _c2m(p):
    p = lax.rem(p + 2 * N_DEV, N_DEV)
    c = p // 4
    i = lax.rem(p, 4)
    z = jnp.where(lax.rem(c, 2) == 0, i, 3 - i)
    return 4 * z + c


def _ag_body(x_ref, w_ref, s_ref, q2d_ref, w_all_ref, s_all_ref,
             fsw, frw, fss, frs, bsw, brw, bss, brs):
    my = lax.axis_index("i")
    qq = lax.rem(my, 4)
    zz = my // 4
    my_p = 4 * qq + jnp.where(lax.rem(qq, 2) == 0, zz, 3 - zz)
    left = _c2m(my_p - 1)
    right = _c2m(my_p + 1)

    barrier = pltpu.get_barrier_semaphore()
    pl.semaphore_signal(barrier, inc=1, device_id=(left,),
                        device_id_type=pl.DeviceIdType.MESH)
    pl.semaphore_signal(barrier, inc=1, device_id=(right,),
                        device_id_type=pl.DeviceIdType.MESH)
    pl.semaphore_wait(barrier, 2)

    w_all_ref[pl.ds(my * CROWS, CROWS), :] = w_ref[...]
    s_all_ref[pl.ds(my, 1), :] = s_ref[...]

    HALF = CROWS // 2

    def w_rdma(p, ss, rs, h, sub, target):
        sl = w_all_ref.at[pl.ds(_c2m(p) * CROWS + sub * HALF, HALF), :]
        return pltpu.make_async_remote_copy(
            src_ref=sl, dst_ref=sl, send_sem=ss.at[h, sub],
            recv_sem=rs.at[h, sub],
            device_id=(target,), device_id_type=pl.DeviceIdType.MESH)

    def s_rdma(p, ss, rs, h, target):
        sl = s_all_ref.at[pl.ds(_c2m(p), 1), :]
        return pltpu.make_async_remote_copy(
            src_ref=sl, dst_ref=sl, send_sem=ss.at[h], recv_sem=rs.at[h],
            device_id=(target,), device_id_type=pl.DeviceIdType.MESH)

    sends = []
    for h in range(FWD):
        for sub in (0, 1):
            if h > 0:
                w_rdma(my_p - h, fsw, frw, h - 1, sub, left).wait_recv()
            s = w_rdma(my_p - h, fsw, frw, h, sub, right)
            s.start()
            sends.append(s)
            if h < BWD:
                if h > 0:
                    w_rdma(my_p + h, bsw, brw, h - 1, sub, right).wait_recv()
                s = w_rdma(my_p + h, bsw, brw, h, sub, left)
                s.start()
                sends.append(s)
        if h > 0:
            s_rdma(my_p - h, fss, frs, h - 1, left).wait_recv()
        s = s_rdma(my_p - h, fss, frs, h, right)
        s.start()
        sends.append(s)
        if h < BWD:
            if h > 0:
                s_rdma(my_p + h, bss, brs, h - 1, right).wait_recv()
            s = s_rdma(my_p + h, bss, brs, h, left)
            s.start()
            sends.append(s)
    for sub in (0, 1):
        w_rdma(my_p - FWD, fsw, frw, FWD - 1, sub, left).wait_recv()
        w_rdma(my_p + BWD, bsw, brw, BWD - 1, sub, right).wait_recv()
    s_rdma(my_p - FWD, fss, frs, FWD - 1, left).wait_recv()
    s_rdma(my_p + BWD, bss, brs, BWD - 1, right).wait_recv()
    for s in sends:
        s.wait_send()

    x = x_ref[...]
    for c in range(N_DEV):
        wq_c = w_all_ref[CROWS * c:CROWS * c + 256, :].astype(jnp.bfloat16)
        q_c = lax.dot_general(x, wq_c, (((1,), (1,)), ((), ())),
                              preferred_element_type=jnp.float32)
        q_c = q_c * s_all_ref[c:c + 1, 0:256]
        q2d_ref[:, 256 * c:256 * c + 256] = q_c.astype(jnp.bfloat16)


def _ag_and_qproj(x2d, w_i8, s_shard):
    return pl.pallas_call(
        _ag_body,
        out_shape=(
            jax.ShapeDtypeStruct((256, 4096), jnp.bfloat16),
            jax.ShapeDtypeStruct((N_DEV * CROWS, 512), jnp.int8),
            jax.ShapeDtypeStruct((N_DEV, 512), jnp.float32),
        ),
        in_specs=[pl.BlockSpec(memory_space=pltpu.VMEM)] * 3,
        out_specs=[pl.BlockSpec(memory_space=pltpu.VMEM)] * 3,
        scratch_shapes=[
            pltpu.SemaphoreType.DMA((FWD, 2)),
            pltpu.SemaphoreType.DMA((FWD, 2)),
            pltpu.SemaphoreType.DMA((FWD,)),
            pltpu.SemaphoreType.DMA((FWD,)),
            pltpu.SemaphoreType.DMA((BWD, 2)),
            pltpu.SemaphoreType.DMA((BWD, 2)),
            pltpu.SemaphoreType.DMA((BWD,)),
            pltpu.SemaphoreType.DMA((BWD,)),
        ],
        compiler_params=pltpu.CompilerParams(collective_id=0),
    )(x2d, w_i8, s_shard)


def _attn_body(q_ref, k_ref, v_ref, o_ref):
    bi = lax.broadcasted_iota(jnp.int32, (256, 256), 0)
    bj = lax.broadcasted_iota(jnp.int32, (256, 256), 1)
    ok = (bi // 128 == bj // 128) & (lax.rem(bj, 128) // 64 <= lax.rem(bi, 128) // 64)
    s = lax.dot_general(q_ref[...], k_ref[...], (((2,), (2,)), ((1,), (1,))),
                        preferred_element_type=jnp.float32) * 0.125
    s = jnp.where(ok[None], s, -1e9)
    m = jnp.max(s, axis=-1, keepdims=True)
    p = jnp.exp(s - m)
    p = (p / jnp.sum(p, axis=-1, keepdims=True)).astype(jnp.bfloat16)
    o_ref[...] = lax.dot_general(
        p, v_ref[...], (((2,), (0,)), ((0,), (1,))),
        preferred_element_type=jnp.float32).astype(jnp.bfloat16)


def _attention(q_m, k_m, v_m):
    return pl.pallas_call(
        _attn_body,
        grid=(4,),
        out_shape=jax.ShapeDtypeStruct((64, 256, 64), jnp.bfloat16),
        in_specs=[
            pl.BlockSpec((256, 16, 64), lambda g: (0, g, 0)),
            pl.BlockSpec((256, 16, 64), lambda g: (0, g, 0)),
            pl.BlockSpec((256, 16, 64), lambda g: (0, g, 0)),
        ],
        out_specs=pl.BlockSpec((16, 256, 64), lambda g: (g, 0, 0)),
    )(q_m, k_m, v_m)


def _out_body(c_ref, w_all_ref, s_ref, o_ref):
    acc = jnp.zeros((256, 512), jnp.float32)
    for c in range(N_DEV):
        ctx_c = c_ref[:, 256 * c:256 * c + 256] * s_ref[c:c + 1, 256:512]
        wo_c = w_all_ref[CROWS * c + 256:CROWS * c + 512, :].astype(jnp.bfloat16)
        acc = acc + jnp.dot(ctx_c.astype(jnp.bfloat16), wo_c,
                            preferred_element_type=jnp.float32)
    o_ref[...] = acc


def _out_proj(ctx2d, w_all, s_all):
    return pl.pallas_call(
        _out_body,
        out_shape=jax.ShapeDtypeStruct((256, 512), jnp.float32),
        in_specs=[pl.BlockSpec(memory_space=pltpu.VMEM)] * 3,
        out_specs=pl.BlockSpec(memory_space=pltpu.VMEM),
    )(ctx2d, w_all, s_all)


def kernel(x, Wq, K_ext, V_ext, Wo):
    bf16 = jnp.bfloat16
    my = lax.axis_index("i")

    x2d = x.reshape(256, 512).astype(bf16)
    wqT = Wq.T
    wo = Wo
    sq = jnp.maximum(jnp.max(jnp.abs(wqT), axis=1), 1e-20) / 127.0
    so = jnp.maximum(jnp.max(jnp.abs(wo), axis=1), 1e-20) / 127.0
    wq_i8 = jnp.round(wqT / sq[:, None]).astype(jnp.int8)
    wo_i8 = jnp.round(wo / so[:, None]).astype(jnp.int8)
    w_i8 = jnp.concatenate([wq_i8, wo_i8], axis=0)
    s_shard = jnp.concatenate([sq, so]).reshape(1, 512).astype(jnp.float32)

    b0 = my * 2
    k_m = lax.dynamic_slice_in_dim(K_ext, b0, 2, 0).astype(bf16).reshape(256, 64, 64)
    v_m = lax.dynamic_slice_in_dim(V_ext, b0, 2, 0).astype(bf16).reshape(256, 64, 64)

    q2d, w_all, s_all = _ag_and_qproj(x2d, w_i8, s_shard)
    q_m = q2d.reshape(256, 64, 64)
    ctx_m = _attention(q_m, k_m, v_m)
    ctx2d = ctx_m.transpose(1, 0, 2).reshape(256, 4096)
    out = _out_proj(ctx2d, w_all, s_all)
    return out.reshape(2, 128, 512)


# device time: 65240 ns/iter; 1.3741x vs baseline; 1.3741x over previous
import jax
import jax.numpy as jnp
from jax import lax
from jax.experimental import pallas as pl
from jax.experimental.pallas import tpu as pltpu

N_DEV = 16
CROWS = 512
FWD = N_DEV // 2
BWD = N_DEV - 1 - FWD


def _c2m(p):
    p = lax.rem(p + 2 * N_DEV, N_DEV)
    c = p // 4
    i = lax.rem(p, 4)
    z = jnp.where(lax.rem(c, 2) == 0, i, 3 - i)
    return 4 * z + c


def _ag_body(x_ref, w_ref, s_ref, q2d_ref, w_all_ref, s_all_ref,
             fsw, frw, fss, frs, bsw, brw, bss, brs):
    my = lax.axis_index("i")
    qq = lax.rem(my, 4)
    zz = my // 4
    my_p = 4 * qq + jnp.where(lax.rem(qq, 2) == 0, zz, 3 - zz)
    left = _c2m(my_p - 1)
    right = _c2m(my_p + 1)

    barrier = pltpu.get_barrier_semaphore()
    pl.semaphore_signal(barrier, inc=1, device_id=(left,),
                        device_id_type=pl.DeviceIdType.MESH)
    pl.semaphore_signal(barrier, inc=1, device_id=(right,),
                        device_id_type=pl.DeviceIdType.MESH)
    pl.semaphore_wait(barrier, 2)

    w_all_ref[pl.ds(my * CROWS, CROWS), :] = w_ref[...]
    s_all_ref[pl.ds(my, 1), :] = s_ref[...]

    NSUB = 4
    SUBROWS = CROWS // NSUB

    def w_rdma(p, ss, rs, h, sub, target):
        sl = w_all_ref.at[pl.ds(_c2m(p) * CROWS + sub * SUBROWS, SUBROWS), :]
        return pltpu.make_async_remote_copy(
            src_ref=sl, dst_ref=sl, send_sem=ss.at[h, sub],
            recv_sem=rs.at[h, sub],
            device_id=(target,), device_id_type=pl.DeviceIdType.MESH)

    def s_rdma(p, ss, rs, h, target):
        sl = s_all_ref.at[pl.ds(_c2m(p), 1), :]
        return pltpu.make_async_remote_copy(
            src_ref=sl, dst_ref=sl, send_sem=ss.at[h], recv_sem=rs.at[h],
            device_id=(target,), device_id_type=pl.DeviceIdType.MESH)

    sends = []
    for h in range(FWD):
        for sub in range(NSUB):
            if h > 0:
                w_rdma(my_p - h, fsw, frw, h - 1, sub, left).wait_recv()
            s = w_rdma(my_p - h, fsw, frw, h, sub, right)
            s.start()
            sends.append(s)
            if h < BWD:
                if h > 0:
                    w_rdma(my_p + h, bsw, brw, h - 1, sub, right).wait_recv()
                s = w_rdma(my_p + h, bsw, brw, h, sub, left)
                s.start()
                sends.append(s)
        if h > 0:
            s_rdma(my_p - h, fss, frs, h - 1, left).wait_recv()
        s = s_rdma(my_p - h, fss, frs, h, right)
        s.start()
        sends.append(s)
        if h < BWD:
            if h > 0:
                s_rdma(my_p + h, bss, brs, h - 1, right).wait_recv()
            s = s_rdma(my_p + h, bss, brs, h, left)
            s.start()
            sends.append(s)
    for sub in range(NSUB):
        w_rdma(my_p - FWD, fsw, frw, FWD - 1, sub, left).wait_recv()
        w_rdma(my_p + BWD, bsw, brw, BWD - 1, sub, right).wait_recv()
    s_rdma(my_p - FWD, fss, frs, FWD - 1, left).wait_recv()
    s_rdma(my_p + BWD, bss, brs, BWD - 1, right).wait_recv()
    for s in sends:
        s.wait_send()

    x = x_ref[...]
    for c in range(N_DEV):
        wq_c = w_all_ref[CROWS * c:CROWS * c + 256, :].astype(jnp.bfloat16)
        q_c = lax.dot_general(x, wq_c, (((1,), (1,)), ((), ())),
                              preferred_element_type=jnp.float32)
        q_c = q_c * s_all_ref[c:c + 1, 0:256]
        q2d_ref[:, 256 * c:256 * c + 256] = q_c.astype(jnp.bfloat16)


def _ag_and_qproj(x2d, w_i8, s_shard):
    return pl.pallas_call(
        _ag_body,
        out_shape=(
            jax.ShapeDtypeStruct((256, 4096), jnp.bfloat16),
            jax.ShapeDtypeStruct((N_DEV * CROWS, 512), jnp.int8),
            jax.ShapeDtypeStruct((N_DEV, 512), jnp.float32),
        ),
        in_specs=[pl.BlockSpec(memory_space=pltpu.VMEM)] * 3,
        out_specs=[pl.BlockSpec(memory_space=pltpu.VMEM)] * 3,
        scratch_shapes=[
            pltpu.SemaphoreType.DMA((FWD, 4)),
            pltpu.SemaphoreType.DMA((FWD, 4)),
            pltpu.SemaphoreType.DMA((FWD,)),
            pltpu.SemaphoreType.DMA((FWD,)),
            pltpu.SemaphoreType.DMA((BWD, 4)),
            pltpu.SemaphoreType.DMA((BWD, 4)),
            pltpu.SemaphoreType.DMA((BWD,)),
            pltpu.SemaphoreType.DMA((BWD,)),
        ],
        compiler_params=pltpu.CompilerParams(collective_id=0),
    )(x2d, w_i8, s_shard)


def _attn_body(q_ref, k_ref, v_ref, o_ref):
    row = lax.broadcasted_iota(jnp.int32, (128, 128), 0) // 64
    col = lax.broadcasted_iota(jnp.int32, (128, 128), 1) // 64
    mask = (col <= row)[None]
    s = lax.dot_general(q_ref[...], k_ref[...], (((2,), (2,)), ((0,), (0,))),
                        preferred_element_type=jnp.float32) * 0.125
    s = jnp.where(mask, s, -1e9)
    m = jnp.max(s, axis=-1, keepdims=True)
    p = jnp.exp(s - m)
    p = p / jnp.sum(p, axis=-1, keepdims=True)
    o_ref[...] = lax.dot_general(
        p.astype(jnp.bfloat16), v_ref[...], (((2,), (1,)), ((0,), (0,))),
        preferred_element_type=jnp.float32).astype(jnp.bfloat16)


def _attention(q3, k3, v3):
    return pl.pallas_call(
        _attn_body,
        grid=(4,),
        out_shape=jax.ShapeDtypeStruct((128, 128, 64), jnp.bfloat16),
        in_specs=[pl.BlockSpec((32, 128, 64), lambda g: (g, 0, 0))] * 3,
        out_specs=pl.BlockSpec((32, 128, 64), lambda g: (g, 0, 0)),
    )(q3, k3, v3)


def _out_body(c_ref, w_all_ref, s_ref, o_ref):
    acc = jnp.zeros((256, 512), jnp.float32)
    for c in range(N_DEV):
        ctx_c = c_ref[:, 256 * c:256 * c + 256] * s_ref[c:c + 1, 256:512]
        wo_c = w_all_ref[CROWS * c + 256:CROWS * c + 512, :].astype(jnp.bfloat16)
        acc = acc + jnp.dot(ctx_c.astype(jnp.bfloat16), wo_c,
                            preferred_element_type=jnp.float32)
    o_ref[...] = acc


def _out_proj(ctx2d, w_all, s_all):
    return pl.pallas_call(
        _out_body,
        out_shape=jax.ShapeDtypeStruct((256, 512), jnp.float32),
        in_specs=[pl.BlockSpec(memory_space=pltpu.VMEM)] * 3,
        out_specs=pl.BlockSpec(memory_space=pltpu.VMEM),
    )(ctx2d, w_all, s_all)


def kernel(x, Wq, K_ext, V_ext, Wo):
    bf16 = jnp.bfloat16
    my = lax.axis_index("i")

    x2d = x.reshape(256, 512).astype(bf16)
    wqT = Wq.T
    wo = Wo
    sq = jnp.maximum(jnp.max(jnp.abs(wqT), axis=1), 1e-20) / 127.0
    so = jnp.maximum(jnp.max(jnp.abs(wo), axis=1), 1e-20) / 127.0
    wq_i8 = jnp.round(wqT / sq[:, None]).astype(jnp.int8)
    wo_i8 = jnp.round(wo / so[:, None]).astype(jnp.int8)
    w_i8 = jnp.concatenate([wq_i8, wo_i8], axis=0)
    s_shard = jnp.concatenate([sq, so]).reshape(1, 512).astype(jnp.float32)

    b0 = my * 2
    k3 = jnp.transpose(
        lax.dynamic_slice_in_dim(K_ext, b0, 2, 0),
        (0, 2, 1, 3)).astype(bf16).reshape(128, 128, 64)
    v3 = jnp.transpose(
        lax.dynamic_slice_in_dim(V_ext, b0, 2, 0),
        (0, 2, 1, 3)).astype(bf16).reshape(128, 128, 64)

    q2d, w_all, s_all = _ag_and_qproj(x2d, w_i8, s_shard)
    q3 = q2d.reshape(2, 128, 64, 64).transpose(0, 2, 1, 3).reshape(128, 128, 64)
    ctx3 = _attention(q3, k3, v3)
    ctx2d = ctx3.reshape(2, 64, 128, 64).transpose(0, 2, 1, 3).reshape(256, 4096)
    out = _out_proj(ctx2d, w_all, s_all)
    return out.reshape(2, 128, 512)
